# baseline (speedup 1.0000x reference)
import functools

import jax
import jax.numpy as jnp
from jax import lax
from jax.experimental import pallas as pl
from jax.experimental.pallas import tpu as pltpu
from jax.experimental.pallas import tpu_sc as plsc

BATCH = 16384
EMBED_DIM = 16
NUM_WORKERS = 32
B_PER_W = BATCH // NUM_WORKERS
GROUPS = B_PER_W // 16

_mesh = plsc.VectorSubcoreMesh(core_axis_name="c", subcore_axis_name="s")


@functools.partial(
    pl.kernel,
    mesh=_mesh,
    out_type=jax.ShapeDtypeStruct((BATCH,), jnp.float32),
    scratch_types=[
        pltpu.VMEM((B_PER_W, 2), jnp.int32),
        pltpu.VMEM((B_PER_W // 128, 128), jnp.int32),
        pltpu.VMEM((B_PER_W // 128, 128), jnp.int32),
        pltpu.VMEM((B_PER_W, EMBED_DIM), jnp.float32),
        pltpu.VMEM((B_PER_W, EMBED_DIM), jnp.float32),
        pltpu.VMEM((B_PER_W,), jnp.float32),
        pltpu.VMEM((1,), jnp.float32),
        pltpu.SemaphoreType.DMA,
        pltpu.SemaphoreType.DMA,
    ],
    compiler_params=pltpu.CompilerParams(
        needs_layout_passes=False, use_tc_tiling_on_sc=False),
)
def _mf_kernel(pairs_hbm, utab_hbm, itab_hbm, bias_hbm, out_hbm,
               pairs_v, uidx_v, iidx_v, urows_v, irows_v, out_v, bias_v,
               sem_u, sem_i):
    wid = lax.axis_index("s") * 2 + lax.axis_index("c")
    base = wid * B_PER_W

    pltpu.sync_copy(pairs_hbm.at[pl.ds(base, B_PER_W)], pairs_v)
    pltpu.sync_copy(bias_hbm, bias_v)

    iota = lax.iota(jnp.int32, 16)
    zeros16 = jnp.zeros((16,), jnp.int32)
    ones16 = jnp.ones((16,), jnp.int32)

    for g in range(GROUPS):
        rows = g * 16 + iota
        c, off = divmod(g * 16, 128)
        uidx_v[c, pl.ds(off, 16)] = plsc.load_gather(pairs_v, [rows, zeros16])
        iidx_v[c, pl.ds(off, 16)] = plsc.load_gather(pairs_v, [rows, ones16])

    copies = []
    for c in range(B_PER_W // 128):
        dst = pl.ds(c * 128, 128)
        copies.append(pltpu.make_async_copy(
            utab_hbm.at[uidx_v.at[c]], urows_v.at[dst], sem_u))
        copies.append(pltpu.make_async_copy(
            itab_hbm.at[iidx_v.at[c]], irows_v.at[dst], sem_i))
    for cp in copies:
        cp.start()
    for cp in copies:
        cp.wait()

    bias_vec = plsc.load_gather(bias_v, [zeros16])

    def dot_body(g, carry):
        rows = g * 16 + iota
        acc = bias_vec
        for d in range(EMBED_DIM):
            cols = jnp.full((16,), d, jnp.int32)
            u = plsc.load_gather(urows_v, [rows, cols])
            v = plsc.load_gather(irows_v, [rows, cols])
            acc = acc + u * v
        out_v[pl.ds(g * 16, 16)] = acc
        return carry

    lax.fori_loop(0, GROUPS, dot_body, 0)

    pltpu.sync_copy(out_v, out_hbm.at[pl.ds(base, B_PER_W)])


def kernel(sparse_inputs, user_table, item_table, bias):
    pairs = sparse_inputs.astype(jnp.int32)
    return _mf_kernel(pairs, user_table, item_table, bias)
